# Initial kernel scaffold; baseline (speedup 1.0000x reference)
import functools

import jax
import jax.numpy as jnp
from jax import lax
from jax.experimental import pallas as pl
from jax.experimental.pallas import tpu as pltpu
from jax.experimental.pallas import tpu_sc as plsc

Q, L, D, K, TOPK = 1024, 10, 512, 100000, 10
NCAND = 2 * TOPK
KBLK = 2048
NKB = (K + KBLK - 1) // KBLK
KPAD = NKB * KBLK
GRP = 128
NGRP = KPAD // GRP
NGV = NGRP // 16
NEG = -1e30
BIGI = jnp.int32(2**30)
BQ = 128

NC, NS, LN = 2, 16, 16
NW = NC * NS
QPW = Q // NW
CROWS = 64


def _ln(x, g, b):
    m = jnp.mean(x, axis=-1, keepdims=True)
    v = jnp.mean((x - m) ** 2, axis=-1, keepdims=True)
    return (x - m) / jnp.sqrt(v + 1e-5) * g + b


def _enc_kernel(ctx_ref, q_ref, wctx_ref, w1_ref, b1_ref, lng_ref, lnb_ref,
                w2_ref, b2_ref, rrw1_ref, rrb1_ref, eq_ref, pre_ref,
                qp_scr, cv_scr):
    l = pl.program_id(1)
    x = ctx_ref[...][:, 0, :]
    qb = q_ref[...]

    @pl.when(l == 0)
    def _():
        qp_scr[...] = jnp.dot(qb, w1_ref[D:, :], preferred_element_type=jnp.float32)
        cv_scr[...] = jnp.zeros_like(cv_scr)
        eq_ref[...] = jnp.zeros_like(eq_ref)

    cv_scr[...] += x * (1.0 / L)
    p = jax.nn.gelu(jnp.dot(x, wctx_ref[...], preferred_element_type=jnp.float32))
    h = jnp.dot(p, w1_ref[:D, :], preferred_element_type=jnp.float32) \
        + qp_scr[...] + b1_ref[...]
    h = jax.nn.gelu(_ln(h, lng_ref[...], lnb_ref[...]))
    caq = jnp.dot(h, w2_ref[...], preferred_element_type=jnp.float32) + b2_ref[...]
    eq_ref[...] += caq * (1.0 / L)

    @pl.when(l == L - 1)
    def _():
        pre_ref[...] = (
            jnp.dot(qb, rrw1_ref[D:2 * D, :], preferred_element_type=jnp.float32)
            + jnp.dot(cv_scr[...], rrw1_ref[2 * D:, :], preferred_element_type=jnp.float32)
            + rrb1_ref[...])


def _score_kernel(eq_ref, keys_ref, s_ref, m_ref):
    kb = pl.program_id(0)
    s = lax.dot_general(eq_ref[...], keys_ref[...], (((1,), (1,)), ((), ())),
                        preferred_element_type=jnp.float32)
    col = kb * KBLK + lax.broadcasted_iota(jnp.int32, (Q, KBLK), 1)
    s = jnp.where(col < K, s, NEG)
    s_ref[...] = s
    m_ref[...] = jnp.max(s.reshape(Q, KBLK // GRP, GRP), axis=2)


def _sel_kernel(srows_hbm, gmax_hbm, keys_hbm, cidx_hbm, cemb_hbm,
                gmax_v, ids_v, chunk_v, hmax_v, run_s, run_i, mrg_s, mrg_i,
                krows_v, sem):
    wid = lax.axis_index("s") * NC + lax.axis_index("c")
    iot = lax.iota(jnp.int32, LN)
    lane0 = iot == 0
    negv = jnp.full((LN,), NEG, jnp.float32)
    zeroi = jnp.zeros((LN,), jnp.int32)

    for i in range(ids_v.shape[0] // LN):
        ids_v[pl.ds(i * LN, LN)] = zeroi
    run_i[pl.ds(0, LN)] = zeroi
    run_i[pl.ds(LN, LN)] = zeroi

    def per_query(j, _carry):
        q = wid * QPW + j
        pltpu.sync_copy(gmax_hbm.at[q], gmax_v)

        def ins_body(i, tops):
            cur = gmax_v[pl.ds(i * LN, LN)]
            out = []
            for _r in range(NCAND):
                hi = jnp.maximum(tops[_r], cur)
                cur = jnp.minimum(tops[_r], cur)
                out.append(hi)
            return tuple(out)

        tops = lax.fori_loop(0, NGV, ins_body,
                             tuple(negv for _ in range(NCAND)))
        thr = jnp.max(tops[NCAND - 1])
        thrv = jnp.full((LN,), thr, jnp.float32)

        def col_body(i, n):
            v = gmax_v[pl.ds(i * LN, LN)]
            m = v >= thrv
            gids = jnp.full((LN,), q * NGRP + i * LN, jnp.int32) + iot
            plsc.store_compressed(ids_v.at[pl.ds(n, LN)], gids, mask=m)
            return n + jnp.sum(m.astype(jnp.int32))

        n = lax.fori_loop(0, NGV, col_body, jnp.int32(0))

        run_s[pl.ds(0, LN)] = negv
        run_s[pl.ds(LN, LN)] = negv

        def chunk_body(c, _c2):
            base = c * CROWS
            rv = jnp.minimum(n - base, CROWS)
            descs = []
            for s4 in range(CROWS // LN):
                iv = ids_v[pl.ds(base + s4 * LN, LN)]
                descs.append(pltpu.async_copy(
                    srows_hbm.at[iv], chunk_v.at[pl.ds(s4 * LN, LN)], sem))
            for dsc in descs:
                dsc.wait()

            for s5 in range(hmax_v.shape[0] // LN):
                hmax_v[pl.ds(s5 * LN, LN)] = negv

            def hx(r, _h):
                mv = negv
                for s8 in range(GRP // LN):
                    mv = jnp.maximum(mv, chunk_v[r, pl.ds(s8 * LN, LN)])
                plsc.store_compressed(
                    hmax_v.at[pl.ds(r, LN)],
                    jnp.full((LN,), jnp.max(mv), jnp.float32), mask=lane0)
                return 0

            lax.fori_loop(0, rv, hx, 0)

            mrg_s[pl.ds(0, LN)] = run_s[pl.ds(0, LN)]
            mrg_s[pl.ds(LN, LN)] = run_s[pl.ds(LN, LN)]
            mrg_i[pl.ds(0, LN)] = run_i[pl.ds(0, LN)]
            mrg_i[pl.ds(LN, LN)] = run_i[pl.ds(LN, LN)]

            def ext_body(e, carry):
                psv, piv = carry
                hm = [hmax_v[pl.ds(v5 * LN, LN)] for v5 in range(5)]
                mm = hm[0]
                for v5 in range(1, 5):
                    mm = jnp.maximum(mm, hm[v5])
                sv = jnp.full((LN,), jnp.max(mm), jnp.float32)
                rr = jnp.full((LN,), BIGI, jnp.int32)
                for v5 in range(5):
                    rio = iot + v5 * LN
                    rr = jnp.minimum(rr, jnp.where(hm[v5] == sv, rio, BIGI))
                rstar = jnp.min(rr)
                rowid = ids_v[base + rstar]
                gb = (rowid - q * NGRP) * GRP
                vs, gis = [], []
                ii = jnp.full((LN,), BIGI, jnp.int32)
                for s8 in range(GRP // LN):
                    v = chunk_v[rstar, pl.ds(s8 * LN, LN)]
                    gi = jnp.full((LN,), gb + s8 * LN, jnp.int32) + iot
                    live = (v < psv) | ((v == psv) & (gi > piv))
                    hit = live & (v == sv)
                    ii = jnp.minimum(ii, jnp.where(hit, gi, BIGI))
                    vs.append(v)
                    gis.append(gi)
                giv = jnp.full((LN,), jnp.min(ii), jnp.int32)
                nm = negv
                for s8 in range(GRP // LN):
                    nl = (vs[s8] < sv) | ((vs[s8] == sv) & (gis[s8] > giv))
                    nm = jnp.maximum(nm, jnp.where(nl, vs[s8], NEG))
                plsc.store_compressed(
                    hmax_v.at[pl.ds(rstar, LN)],
                    jnp.full((LN,), jnp.max(nm), jnp.float32), mask=lane0)
                plsc.store_compressed(mrg_s.at[pl.ds(32 + e, LN)], sv, mask=lane0)
                plsc.store_compressed(mrg_i.at[pl.ds(32 + e, LN)], giv, mask=lane0)
                return sv, giv

            lax.fori_loop(0, NCAND, ext_body,
                          (jnp.full((LN,), 3e38, jnp.float32),
                           jnp.full((LN,), -1, jnp.int32)))

            plsc.store_compressed(mrg_s.at[pl.ds(32 + NCAND, LN)], negv,
                                  mask=iot < 12)
            mrg_s[pl.ds(64, LN)] = negv

            def mex_body(e, carry):
                psv, piv = carry
                mm = negv
                for v5 in range(5):
                    v = mrg_s[pl.ds(v5 * LN, LN)]
                    gi = mrg_i[pl.ds(v5 * LN, LN)]
                    live = (v < psv) | ((v == psv) & (gi > piv))
                    mm = jnp.maximum(mm, jnp.where(live, v, NEG))
                sv = jnp.full((LN,), jnp.max(mm), jnp.float32)
                ii = jnp.full((LN,), BIGI, jnp.int32)
                for v5 in range(5):
                    v = mrg_s[pl.ds(v5 * LN, LN)]
                    gi = mrg_i[pl.ds(v5 * LN, LN)]
                    live = (v < psv) | ((v == psv) & (gi > piv))
                    hit = live & (v == sv)
                    ii = jnp.minimum(ii, jnp.where(hit, gi, BIGI))
                giv = jnp.full((LN,), jnp.min(ii), jnp.int32)
                plsc.store_compressed(run_s.at[pl.ds(e, LN)], sv, mask=lane0)
                plsc.store_compressed(run_i.at[pl.ds(e, LN)], giv, mask=lane0)
                return sv, giv

            lax.fori_loop(0, NCAND, mex_body,
                          (jnp.full((LN,), 3e38, jnp.float32),
                           jnp.full((LN,), -1, jnp.int32)))
            return 0

        nch = (n + CROWS - 1) // CROWS
        lax.fori_loop(0, nch, chunk_body, 0)

        pltpu.sync_copy(run_i.at[pl.ds(0, 24)], cidx_hbm.at[q])

        i0 = run_i[pl.ds(0, LN)]
        i1 = run_i[pl.ds(LN, LN)]
        d0 = pltpu.async_copy(keys_hbm.at[i0], krows_v.at[pl.ds(0, LN)], sem)
        d1 = pltpu.async_copy(keys_hbm.at[i1], krows_v.at[pl.ds(LN, LN)], sem)
        d0.wait()
        d1.wait()
        pltpu.sync_copy(krows_v.at[pl.ds(0, NCAND)],
                        cemb_hbm.at[pl.ds(q * NCAND, NCAND)])
        return 0

    lax.fori_loop(0, QPW, per_query, 0)


def _rr_kernel(cemb_ref, pre_ref, w1a_ref, lng_ref, lnb_ref, w2_ref, b2_ref,
               cidT_ref, fs_ref, fi_ref, rel_scr):
    c = pl.program_id(1)
    x = cemb_ref[...][:, 0, :]
    h = jnp.dot(x, w1a_ref[...], preferred_element_type=jnp.float32) + pre_ref[...]
    h = jax.nn.gelu(_ln(h, lng_ref[...], lnb_ref[...]))
    r = jnp.sum(h * w2_ref[...], axis=1) + b2_ref[0, 0]
    rel_scr[pl.ds(c, 1), :] = jax.nn.sigmoid(r)[None, :]

    @pl.when(c == NCAND - 1)
    def _():
        relm = rel_scr[...]
        ci = cidT_ref[...]
        pos = lax.broadcasted_iota(jnp.int32, (NCAND, BQ), 0)
        cur = relm
        for j in range(TOPK):
            m = jnp.max(cur, axis=0, keepdims=True)
            pstar = jnp.min(jnp.where(cur == m, pos, NCAND), axis=0,
                            keepdims=True)
            sel = pos == pstar
            fs_ref[j:j + 1, :] = m
            fi_ref[j:j + 1, :] = jnp.sum(jnp.where(sel, ci, 0), axis=0,
                                         keepdims=True)
            cur = jnp.where(sel, -1.0, cur)


def kernel(query, context, keys, W_ctx, fusion_W1, fusion_b1, fusion_ln_g,
           fusion_ln_b, fusion_W2, fusion_b2, rr_W1, rr_b1, rr_ln_g, rr_ln_b,
           rr_W2, rr_b2, top_k):
    f32 = jnp.float32
    b1r = fusion_b1.reshape(1, D)
    lngr = fusion_ln_g.reshape(1, D)
    lnbr = fusion_ln_b.reshape(1, D)
    b2r = fusion_b2.reshape(1, D)
    rrb1r = rr_b1.reshape(1, D)
    rlngr = rr_ln_g.reshape(1, D)
    rlnbr = rr_ln_b.reshape(1, D)
    rw2r = rr_W2.reshape(1, D)
    rb2r = rr_b2.reshape(1, 1)

    eq, pre = pl.pallas_call(
        _enc_kernel,
        grid=(Q // BQ, L),
        in_specs=[
            pl.BlockSpec((BQ, 1, D), lambda i, l: (i, l, 0)),
            pl.BlockSpec((BQ, D), lambda i, l: (i, 0)),
            pl.BlockSpec((D, D), lambda i, l: (0, 0)),
            pl.BlockSpec((2 * D, D), lambda i, l: (0, 0)),
            pl.BlockSpec((1, D), lambda i, l: (0, 0)),
            pl.BlockSpec((1, D), lambda i, l: (0, 0)),
            pl.BlockSpec((1, D), lambda i, l: (0, 0)),
            pl.BlockSpec((D, D), lambda i, l: (0, 0)),
            pl.BlockSpec((1, D), lambda i, l: (0, 0)),
            pl.BlockSpec((3 * D, D), lambda i, l: (0, 0)),
            pl.BlockSpec((1, D), lambda i, l: (0, 0)),
        ],
        out_specs=[
            pl.BlockSpec((BQ, D), lambda i, l: (i, 0)),
            pl.BlockSpec((BQ, D), lambda i, l: (i, 0)),
        ],
        out_shape=[
            jax.ShapeDtypeStruct((Q, D), f32),
            jax.ShapeDtypeStruct((Q, D), f32),
        ],
        scratch_shapes=[pltpu.VMEM((BQ, D), f32), pltpu.VMEM((BQ, D), f32)],
    )(context, query, W_ctx, fusion_W1, b1r, lngr, lnbr, fusion_W2, b2r,
      rr_W1, rrb1r)

    scores, gmax = pl.pallas_call(
        _score_kernel,
        grid=(NKB,),
        in_specs=[
            pl.BlockSpec((Q, D), lambda k: (0, 0)),
            pl.BlockSpec((KBLK, D), lambda k: (k, 0)),
        ],
        out_specs=[
            pl.BlockSpec((Q, KBLK), lambda k: (0, k)),
            pl.BlockSpec((Q, KBLK // GRP), lambda k: (0, k)),
        ],
        out_shape=[
            jax.ShapeDtypeStruct((Q, KPAD), f32),
            jax.ShapeDtypeStruct((Q, NGRP), f32),
        ],
    )(eq, keys)

    srows = scores.reshape(Q * NGRP, GRP)

    sel = pl.kernel(
        _sel_kernel,
        out_type=[
            jax.ShapeDtypeStruct((Q, 24), jnp.int32),
            jax.ShapeDtypeStruct((Q * NCAND, D), f32),
        ],
        mesh=plsc.VectorSubcoreMesh(core_axis_name="c", subcore_axis_name="s"),
        scratch_types=[
            pltpu.VMEM((NGRP,), f32),
            pltpu.VMEM((NGRP + LN,), jnp.int32),
            pltpu.VMEM((CROWS, GRP), f32),
            pltpu.VMEM((80,), f32),
            pltpu.VMEM((48,), f32),
            pltpu.VMEM((48,), jnp.int32),
            pltpu.VMEM((80,), f32),
            pltpu.VMEM((80,), jnp.int32),
            pltpu.VMEM((2 * LN, D), f32),
            pltpu.SemaphoreType.DMA,
        ],
    )
    cidx24, cemb = sel(srows, gmax, keys)

    cidT = cidx24[:, :NCAND].T
    cemb3 = cemb.reshape(Q, NCAND, D)
    w1a = rr_W1[:D]

    fsT, fiT = pl.pallas_call(
        _rr_kernel,
        grid=(Q // BQ, NCAND),
        in_specs=[
            pl.BlockSpec((BQ, 1, D), lambda i, c: (i, c, 0)),
            pl.BlockSpec((BQ, D), lambda i, c: (i, 0)),
            pl.BlockSpec((D, D), lambda i, c: (0, 0)),
            pl.BlockSpec((1, D), lambda i, c: (0, 0)),
            pl.BlockSpec((1, D), lambda i, c: (0, 0)),
            pl.BlockSpec((1, D), lambda i, c: (0, 0)),
            pl.BlockSpec((1, 1), lambda i, c: (0, 0)),
            pl.BlockSpec((NCAND, BQ), lambda i, c: (0, i)),
        ],
        out_specs=[
            pl.BlockSpec((TOPK, BQ), lambda i, c: (0, i)),
            pl.BlockSpec((TOPK, BQ), lambda i, c: (0, i)),
        ],
        out_shape=[
            jax.ShapeDtypeStruct((TOPK, Q), f32),
            jax.ShapeDtypeStruct((TOPK, Q), jnp.int32),
        ],
        scratch_shapes=[pltpu.VMEM((NCAND, BQ), f32)],
    )(cemb3, pre, w1a, rlngr, rlnbr, rw2r, rb2r, cidT)

    return fsT.T, fiT.T


# speedup vs baseline: 4.4499x; 4.4499x over previous
import jax
import jax.numpy as jnp
from jax import lax
from jax.experimental import pallas as pl
from jax.experimental.pallas import tpu as pltpu
from jax.experimental.pallas import tpu_sc as plsc

Q, L, D, K, TOPK = 1024, 10, 512, 100000, 10
NCAND = 2 * TOPK
KBLK = 2048
NKB = (K + KBLK - 1) // KBLK
KPAD = NKB * KBLK
GRP = 128
NGRP = KPAD // GRP
NGRP_M = 896
NEG = -1e30
BIGI = 2**30
BQ = 128
NSEL = 32

NC, NS, LN = 2, 16, 16
NW = NC * NS
QPW = Q // NW


def _ln(x, g, b):
    m = jnp.mean(x, axis=-1, keepdims=True)
    v = jnp.mean((x - m) ** 2, axis=-1, keepdims=True)
    return (x - m) / jnp.sqrt(v + 1e-5) * g + b


def _enc_kernel(ctx_ref, q_ref, wctx_ref, w1_ref, b1_ref, lng_ref, lnb_ref,
                w2_ref, b2_ref, eq_ref, cv_ref, acc_scr, cv_scr):
    l = pl.program_id(1)
    x = ctx_ref[0]
    qb = q_ref[...]

    @pl.when(l == 0)
    def _():
        acc_scr[...] = jnp.zeros_like(acc_scr)
        cv_scr[...] = jnp.zeros_like(cv_scr)

    cv_scr[...] += x
    p = jax.nn.gelu(jnp.dot(x, wctx_ref[...], preferred_element_type=jnp.float32))
    comb = jnp.concatenate([p, qb], axis=1)
    h = jnp.dot(comb, w1_ref[...], preferred_element_type=jnp.float32) + b1_ref[...]
    h = jax.nn.gelu(_ln(h, lng_ref[...], lnb_ref[...]))
    caq = jnp.dot(h, w2_ref[...], preferred_element_type=jnp.float32) + b2_ref[...]
    acc_scr[...] += caq

    @pl.when(l == L - 1)
    def _():
        eq_ref[...] = acc_scr[...] / float(L)
        cv_ref[...] = cv_scr[...] / float(L)


def _score_kernel(eq_ref, keys_ref, s_ref, m_ref):
    kb = pl.program_id(0)
    s = lax.dot_general(eq_ref[...], keys_ref[...], (((1,), (1,)), ((), ())),
                        preferred_element_type=jnp.float32)
    col = kb * KBLK + lax.broadcasted_iota(jnp.int32, (Q, KBLK), 1)
    s = jnp.where(col < K, s, NEG)
    s_ref[...] = s
    mloc = jnp.max(s.reshape(Q, KBLK // GRP, GRP), axis=2)
    sub = kb % 8
    NGB = KBLK // GRP

    @pl.when(sub == 0)
    def _():
        m_ref[...] = jnp.full((Q, 8 * NGB), NEG, jnp.float32)
        m_ref[:, 0:NGB] = mloc

    for _t in range(1, 8):
        @pl.when(sub == _t)
        def _(t=_t):
            m_ref[:, t * NGB:(t + 1) * NGB] = mloc


def _gsel_kernel(gmax_ref, gsel_ref):
    cur = gmax_ref[...]
    gpos = lax.broadcasted_iota(jnp.int32, (BQ, NGRP_M), 1)
    g0 = None
    for j in range(NCAND):
        m = jnp.max(cur, axis=1, keepdims=True)
        g = jnp.min(jnp.where(cur == m, gpos, BIGI), axis=1, keepdims=True)
        gsel_ref[:, j:j + 1] = g
        cur = jnp.where(gpos == g, NEG, cur)
        if j == 0:
            g0 = g
    for j in range(NCAND, NSEL):
        gsel_ref[:, j:j + 1] = g0


def _sgath_kernel(srows_hbm, gsel_hbm, cg_hbm, idv, chunk, sem):
    wid = lax.axis_index("s") * NC + lax.axis_index("c")

    def per_query(j, _c):
        q = wid * QPW + j
        pltpu.sync_copy(gsel_hbm.at[pl.ds(q * NSEL, NSEL)], idv)
        base = jnp.full((LN,), q * NGRP, jnp.int32)
        iv0 = idv[pl.ds(0, LN)] + base
        iv1 = idv[pl.ds(LN, LN)] + base
        d0 = pltpu.async_copy(srows_hbm.at[iv0], chunk.at[pl.ds(0, LN)], sem)
        d1 = pltpu.async_copy(srows_hbm.at[iv1], chunk.at[pl.ds(LN, LN)], sem)
        d0.wait()
        d1.wait()
        pltpu.sync_copy(chunk, cg_hbm.at[pl.ds(q * NSEL, NSEL)])
        return 0

    lax.fori_loop(0, QPW, per_query, 0)


def _tsel_kernel(cg_ref, gsel_ref, cidx_ref, gidx_scr):
    io128 = lax.broadcasted_iota(jnp.int32, (BQ, GRP), 1)
    for r in range(NSEL):
        gidx_scr[:, r * GRP:(r + 1) * GRP] = \
            gsel_ref[:, r:r + 1] * GRP + io128
    gidx = gidx_scr[...]
    lane = lax.broadcasted_iota(jnp.int32, (BQ, NSEL * GRP), 1)
    vals = jnp.where(lane < NCAND * GRP, cg_ref[...], NEG)
    c0 = None
    for j in range(NCAND):
        m = jnp.max(vals, axis=1, keepdims=True)
        sel = jnp.min(jnp.where(vals == m, gidx, BIGI), axis=1, keepdims=True)
        cidx_ref[:, j:j + 1] = sel
        vals = jnp.where(gidx == sel, NEG, vals)
        if j == 0:
            c0 = sel
    for j in range(NCAND, NSEL):
        cidx_ref[:, j:j + 1] = c0


def _kgath_kernel(keys_hbm, cidx_hbm, cemb_hbm, idv, krows, sem):
    wid = lax.axis_index("s") * NC + lax.axis_index("c")

    def per_query(j, _c):
        q = wid * QPW + j
        pltpu.sync_copy(cidx_hbm.at[pl.ds(q * NSEL, NSEL)], idv)
        iv0 = idv[pl.ds(0, LN)]
        iv1 = idv[pl.ds(LN, LN)]
        d0 = pltpu.async_copy(keys_hbm.at[iv0], krows.at[pl.ds(0, LN)], sem)
        d1 = pltpu.async_copy(keys_hbm.at[iv1], krows.at[pl.ds(LN, LN)], sem)
        d0.wait()
        d1.wait()
        pltpu.sync_copy(krows.at[pl.ds(0, 24)], cemb_hbm.at[pl.ds(q * 24, 24)])
        return 0

    lax.fori_loop(0, QPW, per_query, 0)


def _rr_kernel(cemb_ref, q_ref, cv_ref, w1_ref, b1_ref, lng_ref, lnb_ref,
               w2_ref, b2_ref, cidT_ref, fs_ref, fi_ref, rel_scr):
    c = pl.program_id(1)
    x = cemb_ref[0]
    inp = jnp.concatenate([x, q_ref[...], cv_ref[...]], axis=1)
    h = jnp.dot(inp, w1_ref[...], preferred_element_type=jnp.float32) + b1_ref[...]
    h = jax.nn.gelu(_ln(h, lng_ref[...], lnb_ref[...]))
    r = jnp.dot(h, w2_ref[...], preferred_element_type=jnp.float32) + b2_ref[...]
    rel_scr[pl.ds(c, 1), :] = jax.nn.sigmoid(r)[:, 0][None, :]

    @pl.when(c == NCAND - 1)
    def _():
        relm = rel_scr[...]
        ci = cidT_ref[...]
        pos = lax.broadcasted_iota(jnp.int32, (NCAND, BQ), 0)
        cur = relm
        for j in range(TOPK):
            m = jnp.max(cur, axis=0, keepdims=True)
            pstar = jnp.min(jnp.where(cur == m, pos, NCAND), axis=0,
                            keepdims=True)
            sel = pos == pstar
            fs_ref[j:j + 1, :] = m
            fi_ref[j:j + 1, :] = jnp.sum(jnp.where(sel, ci, 0), axis=0,
                                         keepdims=True)
            cur = jnp.where(sel, -1.0, cur)


def kernel(query, context, keys, W_ctx, fusion_W1, fusion_b1, fusion_ln_g,
           fusion_ln_b, fusion_W2, fusion_b2, rr_W1, rr_b1, rr_ln_g, rr_ln_b,
           rr_W2, rr_b2, top_k):
    f32 = jnp.float32
    i32 = jnp.int32
    b1r = fusion_b1.reshape(1, D)
    lngr = fusion_ln_g.reshape(1, D)
    lnbr = fusion_ln_b.reshape(1, D)
    b2r = fusion_b2.reshape(1, D)
    rrb1r = rr_b1.reshape(1, D)
    rlngr = rr_ln_g.reshape(1, D)
    rlnbr = rr_ln_b.reshape(1, D)
    rb2r = rr_b2.reshape(1, 1)

    ctx_t = jnp.transpose(context, (1, 0, 2))

    eq, cvec = pl.pallas_call(
        _enc_kernel,
        grid=(Q // BQ, L),
        in_specs=[
            pl.BlockSpec((1, BQ, D), lambda i, l: (l, i, 0)),
            pl.BlockSpec((BQ, D), lambda i, l: (i, 0)),
            pl.BlockSpec((D, D), lambda i, l: (0, 0)),
            pl.BlockSpec((2 * D, D), lambda i, l: (0, 0)),
            pl.BlockSpec((1, D), lambda i, l: (0, 0)),
            pl.BlockSpec((1, D), lambda i, l: (0, 0)),
            pl.BlockSpec((1, D), lambda i, l: (0, 0)),
            pl.BlockSpec((D, D), lambda i, l: (0, 0)),
            pl.BlockSpec((1, D), lambda i, l: (0, 0)),
        ],
        out_specs=[
            pl.BlockSpec((BQ, D), lambda i, l: (i, 0)),
            pl.BlockSpec((BQ, D), lambda i, l: (i, 0)),
        ],
        out_shape=[
            jax.ShapeDtypeStruct((Q, D), f32),
            jax.ShapeDtypeStruct((Q, D), f32),
        ],
        scratch_shapes=[pltpu.VMEM((BQ, D), f32), pltpu.VMEM((BQ, D), f32)],
    )(ctx_t, query, W_ctx, fusion_W1, b1r, lngr, lnbr, fusion_W2, b2r)

    scores, gmax = pl.pallas_call(
        _score_kernel,
        grid=(NKB,),
        in_specs=[
            pl.BlockSpec((Q, D), lambda k: (0, 0)),
            pl.BlockSpec((KBLK, D), lambda k: (k, 0)),
        ],
        out_specs=[
            pl.BlockSpec((Q, KBLK), lambda k: (0, k)),
            pl.BlockSpec((Q, 128), lambda k: (0, k // 8)),
        ],
        out_shape=[
            jax.ShapeDtypeStruct((Q, KPAD), f32),
            jax.ShapeDtypeStruct((Q, NGRP_M), f32),
        ],
    )(eq, keys)

    gsel = pl.pallas_call(
        _gsel_kernel,
        grid=(Q // BQ,),
        in_specs=[pl.BlockSpec((BQ, NGRP_M), lambda i: (i, 0))],
        out_specs=pl.BlockSpec((BQ, NSEL), lambda i: (i, 0)),
        out_shape=jax.ShapeDtypeStruct((Q, NSEL), i32),
    )(gmax)

    srows = scores.reshape(Q * NGRP, GRP)
    gself = gsel.reshape(Q * NSEL)

    sgath = pl.kernel(
        _sgath_kernel,
        out_type=jax.ShapeDtypeStruct((Q * NSEL, GRP), f32),
        mesh=plsc.VectorSubcoreMesh(core_axis_name="c", subcore_axis_name="s"),
        scratch_types=[
            pltpu.VMEM((NSEL,), i32),
            pltpu.VMEM((NSEL, GRP), f32),
            pltpu.SemaphoreType.DMA,
        ],
    )
    cg = sgath(srows, gself)

    cidx = pl.pallas_call(
        _tsel_kernel,
        grid=(Q // BQ,),
        in_specs=[
            pl.BlockSpec((BQ, NSEL * GRP), lambda i: (i, 0)),
            pl.BlockSpec((BQ, NSEL), lambda i: (i, 0)),
        ],
        out_specs=pl.BlockSpec((BQ, NSEL), lambda i: (i, 0)),
        out_shape=jax.ShapeDtypeStruct((Q, NSEL), i32),
        scratch_shapes=[pltpu.VMEM((BQ, NSEL * GRP), i32)],
    )(cg.reshape(Q, NSEL * GRP), gsel)

    kgath = pl.kernel(
        _kgath_kernel,
        out_type=jax.ShapeDtypeStruct((Q * 24, D), f32),
        mesh=plsc.VectorSubcoreMesh(core_axis_name="c", subcore_axis_name="s"),
        scratch_types=[
            pltpu.VMEM((NSEL,), i32),
            pltpu.VMEM((NSEL, D), f32),
            pltpu.SemaphoreType.DMA,
        ],
    )
    cemb = kgath(keys, cidx.reshape(Q * NSEL))

    cidT = cidx[:, :NCAND].T
    cembT = jnp.transpose(cemb.reshape(Q, 24, D)[:, :NCAND], (1, 0, 2))

    fsT, fiT = pl.pallas_call(
        _rr_kernel,
        grid=(Q // BQ, NCAND),
        in_specs=[
            pl.BlockSpec((1, BQ, D), lambda i, c: (c, i, 0)),
            pl.BlockSpec((BQ, D), lambda i, c: (i, 0)),
            pl.BlockSpec((BQ, D), lambda i, c: (i, 0)),
            pl.BlockSpec((3 * D, D), lambda i, c: (0, 0)),
            pl.BlockSpec((1, D), lambda i, c: (0, 0)),
            pl.BlockSpec((1, D), lambda i, c: (0, 0)),
            pl.BlockSpec((1, D), lambda i, c: (0, 0)),
            pl.BlockSpec((D, 1), lambda i, c: (0, 0)),
            pl.BlockSpec((1, 1), lambda i, c: (0, 0)),
            pl.BlockSpec((NCAND, BQ), lambda i, c: (0, i)),
        ],
        out_specs=[
            pl.BlockSpec((TOPK, BQ), lambda i, c: (0, i)),
            pl.BlockSpec((TOPK, BQ), lambda i, c: (0, i)),
        ],
        out_shape=[
            jax.ShapeDtypeStruct((TOPK, Q), f32),
            jax.ShapeDtypeStruct((TOPK, Q), jnp.int32),
        ],
        scratch_shapes=[pltpu.VMEM((NCAND, BQ), f32)],
    )(cembT, query, cvec, rr_W1, rrb1r, rlngr, rlnbr, rr_W2, rb2r, cidT)

    return fsT.T, fiT.T
